# initial kernel scaffold (unmeasured)
import jax
import jax.numpy as jnp
from jax import lax
from jax.experimental import pallas as pl
from jax.experimental.pallas import tpu as pltpu

N_DEV = 8
B, T, C = 2, 4096, 1024
M = B * T
CHUNK = M // N_DEV
CHUNKS_PER_BATCH = T // CHUNK
KTAPS = 4


def _body(x_ref, k_ref, w_ref, out_ref, p_ref, recva_ref, red_ref,
          send_sems, recva_sems, recvb_sems):
    my = lax.axis_index("i")

    barrier = pltpu.get_barrier_semaphore()
    for o in range(1, N_DEV):
        pl.semaphore_signal(
            barrier, inc=1,
            device_id=(lax.rem(my + o, N_DEV),),
            device_id_type=pl.DeviceIdType.MESH,
        )
    pl.semaphore_wait(barrier, N_DEV - 1)

    w = w_ref[...]

    for c in range(N_DEV):
        b, tc = divmod(c, CHUNKS_PER_BATCH)
        t0 = tc * CHUNK
        if t0 == 0:
            ctx = jnp.concatenate(
                [jnp.zeros((KTAPS - 1, C), jnp.float32),
                 x_ref[b, 0:CHUNK, :].astype(jnp.float32)],
                axis=0,
            )
        else:
            ctx = x_ref[b, t0 - (KTAPS - 1):t0 + CHUNK, :].astype(jnp.float32)
        y = ctx[KTAPS - 1:KTAPS - 1 + CHUNK, :] * k_ref[KTAPS - 1:KTAPS, :]
        for j in range(KTAPS - 1):
            y += ctx[j:j + CHUNK, :] * k_ref[j:j + 1, :]
        a = (y * jax.nn.sigmoid(y)).astype(jnp.bfloat16)
        p_ref[c * CHUNK:(c + 1) * CHUNK, :] = jnp.dot(
            a, w, preferred_element_type=jnp.float32
        ).astype(jnp.bfloat16)

    sends_a = []
    for o in range(1, N_DEV):
        dst = lax.rem(my + o, N_DEV)
        rdma = pltpu.make_async_remote_copy(
            src_ref=p_ref.at[pl.ds(dst * CHUNK, CHUNK), :],
            dst_ref=recva_ref.at[o - 1],
            send_sem=send_sems.at[o - 1],
            recv_sem=recva_sems.at[o - 1],
            device_id=(dst,),
            device_id_type=pl.DeviceIdType.MESH,
        )
        rdma.start()
        sends_a.append(rdma)

    red = pl.load(p_ref, (pl.ds(my * CHUNK, CHUNK), slice(None))).astype(
        jnp.float32)
    for o in range(1, N_DEV):
        sends_a[o - 1].wait_recv()
        red = red + recva_ref[o - 1].astype(jnp.float32)
    red_bf = red.astype(jnp.bfloat16)
    red_ref[...] = red_bf
    out_ref[pl.ds(my * CHUNK, CHUNK), :] = red_bf

    for r in sends_a:
        r.wait_send()

    sends_b = []
    for o in range(1, N_DEV):
        dst = lax.rem(my + o, N_DEV)
        rdma = pltpu.make_async_remote_copy(
            src_ref=red_ref,
            dst_ref=out_ref.at[pl.ds(my * CHUNK, CHUNK), :],
            send_sem=send_sems.at[o - 1],
            recv_sem=recvb_sems.at[o - 1],
            device_id=(dst,),
            device_id_type=pl.DeviceIdType.MESH,
        )
        rdma.start()
        sends_b.append(rdma)

    for r in sends_b:
        r.wait_recv()
    for r in sends_b:
        r.wait_send()


def kernel(x, k, Wp):
    xb = x.astype(jnp.bfloat16)
    wb = Wp.astype(jnp.bfloat16)
    kf = k.astype(jnp.float32)

    out_flat = pl.pallas_call(
        _body,
        out_shape=jax.ShapeDtypeStruct((M, C), jnp.bfloat16),
        in_specs=[
            pl.BlockSpec(memory_space=pltpu.VMEM),
            pl.BlockSpec(memory_space=pltpu.VMEM),
            pl.BlockSpec(memory_space=pltpu.VMEM),
        ],
        out_specs=pl.BlockSpec(memory_space=pltpu.VMEM),
        scratch_shapes=[
            pltpu.VMEM((M, C), jnp.bfloat16),
            pltpu.VMEM((N_DEV - 1, CHUNK, C), jnp.bfloat16),
            pltpu.VMEM((CHUNK, C), jnp.bfloat16),
            pltpu.SemaphoreType.DMA((N_DEV - 1,)),
            pltpu.SemaphoreType.DMA((N_DEV - 1,)),
            pltpu.SemaphoreType.DMA((N_DEV - 1,)),
        ],
        compiler_params=pltpu.CompilerParams(collective_id=0),
    )(xb, kf, wb)
    return out_flat.reshape(B, T, C).astype(jnp.float32)


# baseline (device time: 358669 ns/iter reference)
import jax
import jax.numpy as jnp
from jax import lax
from jax.experimental import pallas as pl
from jax.experimental.pallas import tpu as pltpu

N_DEV = 8
B, T, C = 2, 4096, 1024
M = B * T
CHUNK = M // N_DEV
CHUNKS_PER_BATCH = T // CHUNK
KTAPS = 4


def _body(x_ref, k_ref, w_ref, out_ref, recva_ref, red_ref,
          send_sems, recva_sems, recvb_sems):
    my = lax.axis_index("i")

    barrier = pltpu.get_barrier_semaphore()
    for o in range(1, N_DEV):
        pl.semaphore_signal(
            barrier, inc=1,
            device_id=(lax.rem(my + o, N_DEV),),
            device_id_type=pl.DeviceIdType.MESH,
        )
    pl.semaphore_wait(barrier, N_DEV - 1)

    w = w_ref[...]

    for c in range(N_DEV):
        b, tc = divmod(c, CHUNKS_PER_BATCH)
        t0 = tc * CHUNK
        if t0 == 0:
            ctx = jnp.concatenate(
                [jnp.zeros((KTAPS - 1, C), jnp.float32),
                 x_ref[b, 0:CHUNK, :].astype(jnp.float32)],
                axis=0,
            )
        else:
            ctx = x_ref[b, t0 - (KTAPS - 1):t0 + CHUNK, :].astype(jnp.float32)
        y = ctx[KTAPS - 1:KTAPS - 1 + CHUNK, :] * k_ref[KTAPS - 1:KTAPS, :]
        for j in range(KTAPS - 1):
            y += ctx[j:j + CHUNK, :] * k_ref[j:j + 1, :]
        a = (y * jax.nn.sigmoid(y)).astype(jnp.bfloat16)
        out_ref[c * CHUNK:(c + 1) * CHUNK, :] = jnp.dot(
            a, w, preferred_element_type=jnp.float32
        ).astype(jnp.bfloat16)

    sends_a = []
    for o in range(1, N_DEV):
        dst = lax.rem(my + o, N_DEV)
        rdma = pltpu.make_async_remote_copy(
            src_ref=out_ref.at[pl.ds(dst * CHUNK, CHUNK), :],
            dst_ref=recva_ref.at[o - 1],
            send_sem=send_sems.at[o - 1],
            recv_sem=recva_sems.at[o - 1],
            device_id=(dst,),
            device_id_type=pl.DeviceIdType.MESH,
        )
        rdma.start()
        sends_a.append(rdma)

    red = out_ref[pl.ds(my * CHUNK, CHUNK), :].astype(jnp.float32)
    for o in range(1, N_DEV):
        sends_a[o - 1].wait_recv()
        red = red + recva_ref[o - 1].astype(jnp.float32)
    red_bf = red.astype(jnp.bfloat16)
    red_ref[...] = red_bf
    out_ref[pl.ds(my * CHUNK, CHUNK), :] = red_bf

    for r in sends_a:
        r.wait_send()

    sends_b = []
    for o in range(1, N_DEV):
        dst = lax.rem(my + o, N_DEV)
        rdma = pltpu.make_async_remote_copy(
            src_ref=red_ref,
            dst_ref=out_ref.at[pl.ds(my * CHUNK, CHUNK), :],
            send_sem=send_sems.at[o - 1],
            recv_sem=recvb_sems.at[o - 1],
            device_id=(dst,),
            device_id_type=pl.DeviceIdType.MESH,
        )
        rdma.start()
        sends_b.append(rdma)

    for r in sends_b:
        r.wait_recv()
    for r in sends_b:
        r.wait_send()


def kernel(x, k, Wp):
    xb = x.astype(jnp.bfloat16)
    wb = Wp.astype(jnp.bfloat16)
    kf = k.astype(jnp.float32)

    out_flat = pl.pallas_call(
        _body,
        out_shape=jax.ShapeDtypeStruct((M, C), jnp.bfloat16),
        in_specs=[
            pl.BlockSpec(memory_space=pltpu.VMEM),
            pl.BlockSpec(memory_space=pltpu.VMEM),
            pl.BlockSpec(memory_space=pltpu.VMEM),
        ],
        out_specs=pl.BlockSpec(memory_space=pltpu.VMEM),
        scratch_shapes=[
            pltpu.VMEM((N_DEV - 1, CHUNK, C), jnp.bfloat16),
            pltpu.VMEM((CHUNK, C), jnp.bfloat16),
            pltpu.SemaphoreType.DMA((N_DEV - 1,)),
            pltpu.SemaphoreType.DMA((N_DEV - 1,)),
            pltpu.SemaphoreType.DMA((N_DEV - 1,)),
        ],
        compiler_params=pltpu.CompilerParams(
            collective_id=0, vmem_limit_bytes=100 * 1024 * 1024),
    )(xb, kf, wb)
    return out_flat.reshape(B, T, C).astype(jnp.float32)


# device time: 322132 ns/iter; 1.1134x vs baseline; 1.1134x over previous
import jax
import jax.numpy as jnp
from jax import lax
from jax.experimental import pallas as pl
from jax.experimental.pallas import tpu as pltpu

N_DEV = 8
B, T, C = 2, 4096, 1024
M = B * T
CHUNK = M // N_DEV
CHUNKS_PER_BATCH = T // CHUNK
KTAPS = 4


def _body(x_ref, k_ref, w_ref, out_ref, recva_ref, red_ref,
          send_sems, recva_sems, recvb_sems):
    my = lax.axis_index("i")

    barrier = pltpu.get_barrier_semaphore()
    for o in range(1, N_DEV):
        pl.semaphore_signal(
            barrier, inc=1,
            device_id=(lax.rem(my + o, N_DEV),),
            device_id_type=pl.DeviceIdType.MESH,
        )
    pl.semaphore_wait(barrier, N_DEV - 1)

    w = w_ref[...]

    for c in range(N_DEV):
        b, tc = divmod(c, CHUNKS_PER_BATCH)
        t0 = tc * CHUNK
        if t0 == 0:
            ctx = jnp.concatenate(
                [jnp.zeros((KTAPS - 1, C), jnp.float32),
                 x_ref[b, 0:CHUNK, :].astype(jnp.float32)],
                axis=0,
            )
        else:
            ctx = x_ref[b, t0 - (KTAPS - 1):t0 + CHUNK, :].astype(jnp.float32)
        y = ctx[KTAPS - 1:KTAPS - 1 + CHUNK, :] * k_ref[KTAPS - 1:KTAPS, :]
        for j in range(KTAPS - 1):
            y += ctx[j:j + CHUNK, :] * k_ref[j:j + 1, :]
        a = (y * jax.nn.sigmoid(y)).astype(jnp.bfloat16)
        out_ref[c * CHUNK:(c + 1) * CHUNK, :] = jnp.dot(
            a, w, preferred_element_type=jnp.float32
        ).astype(jnp.bfloat16)

        o = lax.rem(jnp.int32(c) - my + N_DEV, N_DEV)

        @pl.when(c != my)
        def _():
            rdma = pltpu.make_async_remote_copy(
                src_ref=out_ref.at[pl.ds(c * CHUNK, CHUNK), :],
                dst_ref=recva_ref.at[o - 1],
                send_sem=send_sems.at[o - 1],
                recv_sem=recva_sems.at[o - 1],
                device_id=(c,),
                device_id_type=pl.DeviceIdType.MESH,
            )
            rdma.start()

    red = out_ref[pl.ds(my * CHUNK, CHUNK), :].astype(jnp.float32)
    for o in range(1, N_DEV):
        recv = pltpu.make_async_remote_copy(
            src_ref=red_ref,
            dst_ref=recva_ref.at[o - 1],
            send_sem=send_sems.at[o - 1],
            recv_sem=recva_sems.at[o - 1],
            device_id=(my,),
            device_id_type=pl.DeviceIdType.MESH,
        )
        recv.wait_recv()
        red = red + recva_ref[o - 1].astype(jnp.float32)
    red_bf = red.astype(jnp.bfloat16)
    red_ref[...] = red_bf
    out_ref[pl.ds(my * CHUNK, CHUNK), :] = red_bf

    for o in range(1, N_DEV):
        drain = pltpu.make_async_remote_copy(
            src_ref=red_ref,
            dst_ref=recva_ref.at[o - 1],
            send_sem=send_sems.at[o - 1],
            recv_sem=recva_sems.at[o - 1],
            device_id=(my,),
            device_id_type=pl.DeviceIdType.MESH,
        )
        drain.wait_send()

    sends_b = []
    for o in range(1, N_DEV):
        dst = lax.rem(my + o, N_DEV)
        rdma = pltpu.make_async_remote_copy(
            src_ref=red_ref,
            dst_ref=out_ref.at[pl.ds(my * CHUNK, CHUNK), :],
            send_sem=send_sems.at[o - 1],
            recv_sem=recvb_sems.at[o - 1],
            device_id=(dst,),
            device_id_type=pl.DeviceIdType.MESH,
        )
        rdma.start()
        sends_b.append(rdma)

    for r in sends_b:
        r.wait_recv()
    for r in sends_b:
        r.wait_send()


def kernel(x, k, Wp):
    xb = x.astype(jnp.bfloat16)
    wb = Wp.astype(jnp.bfloat16)
    kf = k.astype(jnp.float32)

    out_flat = pl.pallas_call(
        _body,
        out_shape=jax.ShapeDtypeStruct((M, C), jnp.bfloat16),
        in_specs=[
            pl.BlockSpec(memory_space=pltpu.VMEM),
            pl.BlockSpec(memory_space=pltpu.VMEM),
            pl.BlockSpec(memory_space=pltpu.VMEM),
        ],
        out_specs=pl.BlockSpec(memory_space=pltpu.VMEM),
        scratch_shapes=[
            pltpu.VMEM((N_DEV - 1, CHUNK, C), jnp.bfloat16),
            pltpu.VMEM((CHUNK, C), jnp.bfloat16),
            pltpu.SemaphoreType.DMA((N_DEV - 1,)),
            pltpu.SemaphoreType.DMA((N_DEV - 1,)),
            pltpu.SemaphoreType.DMA((N_DEV - 1,)),
        ],
        compiler_params=pltpu.CompilerParams(
            collective_id=0, vmem_limit_bytes=100 * 1024 * 1024),
    )(xb, kf, wb)
    return out_flat.reshape(B, T, C).astype(jnp.float32)
